# Initial kernel scaffold; baseline (speedup 1.0000x reference)
import jax
import jax.numpy as jnp
from jax.experimental import pallas as pl


def kernel(x, edge_index, params):
    raise NotImplementedError("write your pallas kernel here")


# speedup vs baseline: 3.7506x; 3.7506x over previous
import functools

import jax
import jax.numpy as jnp
import numpy as np
from jax import lax
from jax.experimental import pallas as pl
from jax.experimental.pallas import tpu as pltpu
from jax.experimental.pallas import tpu_sc as plsc

N = 10000
E = 320000
D = 128
F_OUT = 64
DF = 2 * D
AVG_LOG = float(np.log(33.0))

NC = 2
NS = 16
NW = NC * NS
RPW = 4
NR = NW * RPW
RN = 80
NPAD = NR * RN
ACC_ROWS = RN + 1

CEDGE = 128
FLUSH = 4096
SCH = 8000
GRP = 25
NBUF = FLUSH + 512
CAP = 80 * FLUSH

_SC_PARAMS = pltpu.CompilerParams(needs_layout_passes=False)


def _sc_mesh():
    return plsc.VectorSubcoreMesh(
        core_axis_name="c", subcore_axis_name="s", num_cores=NC, num_subcores=NS
    )


@functools.partial(
    pl.kernel,
    out_type=(
        jax.ShapeDtypeStruct((NR * CAP,), jnp.int32),
        jax.ShapeDtypeStruct((NR * CAP,), jnp.int32),
        jax.ShapeDtypeStruct((NR * 16,), jnp.int32),
    ),
    mesh=_sc_mesh(),
    compiler_params=_SC_PARAMS,
    scratch_types=[
        pltpu.VMEM((SCH,), jnp.int32),
        pltpu.VMEM((SCH,), jnp.int32),
    ] + [pltpu.VMEM((NBUF,), jnp.int32) for _ in range(2 * RPW)] + [
        pltpu.VMEM((16,), jnp.int32),
    ],
)
def _build_lists(dst_hbm, src_hbm, srcl_hbm, dll_hbm, cnt_hbm,
                 std, sts, bs0, bs1, bs2, bs3, bd0, bd1, bd2, bd3, cvec):
    bs = [bs0, bs1, bs2, bs3]
    bd = [bd0, bd1, bd2, bd3]
    wid = lax.axis_index("s") * NC + lax.axis_index("c")
    wbase = wid * (RPW * RN)
    iota = lax.iota(jnp.int32, 16)
    dummy_dl = jnp.full((16,), RN, jnp.int32)

    def chunk_body(ch, carry):
        pltpu.sync_copy(dst_hbm.at[pl.ds(ch * SCH, SCH)], std)
        pltpu.sync_copy(src_hbm.at[pl.ds(ch * SCH, SCH)], sts)

        def grp_body(g, c2):
            offs = list(c2[:RPW])
            goffs = list(c2[RPW:])
            for u in range(GRP):
                sl = pl.ds((g * GRP + u) * 16, 16)
                dv = std[sl]
                sv = sts[sl]
                q = dv - wbase
                for kk in range(RPW):
                    m = (q >= kk * RN) & (q < (kk + 1) * RN)
                    m32 = jnp.where(m, jnp.int32(1), jnp.int32(0))
                    pc = plsc.cumsum(m32)
                    popc = plsc.all_reduce_population_count(m)
                    tgt = offs[kk] + pc - m32
                    plsc.store_scatter(bs[kk], [tgt], sv, mask=m)
                    plsc.store_scatter(bd[kk], [tgt], q - kk * RN, mask=m)
                    offs[kk] = offs[kk] + popc
            for kk in range(RPW):
                r = wid * RPW + kk
                off_s = jnp.max(offs[kk])

                def do_flush(op, kk=kk, r=r):
                    off_v, go = op
                    base = pl.multiple_of(r * CAP + go, 8)
                    pltpu.sync_copy(bs[kk].at[pl.ds(0, FLUSH)],
                                    srcl_hbm.at[pl.ds(base, FLUSH)])
                    pltpu.sync_copy(bd[kk].at[pl.ds(0, FLUSH)],
                                    dll_hbm.at[pl.ds(base, FLUSH)])
                    for t in range((NBUF - FLUSH) // 16):
                        s_src = pl.ds(FLUSH + t * 16, 16)
                        s_dst = pl.ds(t * 16, 16)
                        bs[kk][s_dst] = bs[kk][s_src]
                        bd[kk][s_dst] = bd[kk][s_src]
                    return (off_v - FLUSH, go + FLUSH)

                offs[kk], goffs[kk] = lax.cond(
                    off_s >= FLUSH, do_flush, lambda op: op,
                    (offs[kk], goffs[kk]))
            return tuple(offs) + tuple(goffs)

        return lax.fori_loop(0, SCH // 16 // GRP, grp_body, carry)

    zero_v = jnp.zeros((16,), jnp.int32)
    zero_s = jnp.int32(0)
    carry = (zero_v,) * RPW + (zero_s,) * RPW
    carry = lax.fori_loop(0, E // SCH, chunk_body, carry)

    for kk in range(RPW):
        r = wid * RPW + kk
        off_s = jnp.max(carry[kk])
        go = carry[RPW + kk]
        start = (off_s // 16) * 16
        pos = start + iota
        plsc.store_scatter(bs[kk], [pos], iota, mask=pos >= off_s)
        plsc.store_scatter(bd[kk], [pos], dummy_dl, mask=pos >= off_s)
        for t in range(1, 9):
            bs[kk][pl.ds(start + t * 16, 16)] = iota
            bd[kk][pl.ds(start + t * 16, 16)] = dummy_dl
        base = pl.multiple_of(r * CAP + go, 8)
        pltpu.sync_copy(bs[kk].at[pl.ds(0, FLUSH)],
                        srcl_hbm.at[pl.ds(base, FLUSH)])
        pltpu.sync_copy(bd[kk].at[pl.ds(0, FLUSH)],
                        dll_hbm.at[pl.ds(base, FLUSH)])
        cvec[...] = jnp.full((16,), go + off_s, jnp.int32)
        pltpu.sync_copy(cvec, cnt_hbm.at[pl.ds(pl.multiple_of(r * 16, 8), 16)])


@functools.partial(
    pl.kernel,
    out_type=tuple(
        [jax.ShapeDtypeStruct((NPAD, D), jnp.float32)] * 8
        + [jax.ShapeDtypeStruct((NPAD, 16), jnp.float32)]
    ),
    mesh=_sc_mesh(),
    compiler_params=_SC_PARAMS,
    scratch_types=[
        pltpu.VMEM((ACC_ROWS, D), jnp.float32),
        pltpu.VMEM((ACC_ROWS, D), jnp.float32),
        pltpu.VMEM((ACC_ROWS, D), jnp.float32),
        pltpu.VMEM((ACC_ROWS, D), jnp.float32),
        pltpu.VMEM((ACC_ROWS, 16), jnp.float32),
        pltpu.VMEM((CEDGE, D), jnp.float32),
        pltpu.VMEM((CEDGE,), jnp.int32),
        pltpu.VMEM((CEDGE,), jnp.int32),
        pltpu.VMEM((16,), jnp.int32),
        pltpu.SemaphoreType.DMA,
    ],
)
def _aggregate(tab0, tab1, srcl_hbm, dll_hbm, cnt_hbm,
               s1a, s2a, mna, mxa, s1b, s2b, mnb, mxb, cnto,
               asum, asq, amn, amx, acnt, gbuf, sidx, dlv, cvec, sem):
    wid = lax.axis_index("s") * NC + lax.axis_index("c")
    iota = lax.iota(jnp.int32, 16)
    zero16 = jnp.zeros((16,), jnp.float32)
    pinf = jnp.full((16,), jnp.inf, jnp.float32)
    ninf = jnp.full((16,), -jnp.inf, jnp.float32)
    ones = jnp.ones((16,), jnp.float32)

    for t, (tab, s1, s2, mno, mxo) in enumerate(
        ((tab0, s1a, s2a, mna, mxa), (tab1, s1b, s2b, mnb, mxb))
    ):
        def range_body(kk, _0, tab=tab, s1=s1, s2=s2, mno=mno, mxo=mxo, t=t):
            r = wid * RPW + kk

            def init_row(i, _):
                for j in range(D // 16):
                    sl = pl.ds(j * 16, 16)
                    asum[i, sl] = zero16
                    asq[i, sl] = zero16
                    amn[i, sl] = pinf
                    amx[i, sl] = ninf
                acnt[i, pl.ds(0, 16)] = zero16
                return 0

            lax.fori_loop(0, ACC_ROWS, init_row, 0)

            pltpu.sync_copy(
                cnt_hbm.at[pl.ds(pl.multiple_of(r * 16, 8), 16)], cvec)
            cn = jnp.max(cvec[...])
            nch = (cn + CEDGE - 1) // CEDGE

            def chunk_body(ch, _, tab=tab):
                base = pl.multiple_of(r * CAP + ch * CEDGE, 8)
                pltpu.sync_copy(srcl_hbm.at[pl.ds(base, CEDGE)], sidx)
                pltpu.sync_copy(dll_hbm.at[pl.ds(base, CEDGE)], dlv)
                pltpu.async_copy(tab.at[sidx], gbuf, sem).wait()

                def grp16(g, _2):
                    dls = dlv[pl.ds(g * 16, 16)]

                    def sub4(s4, _3):
                        for e4 in range(4):
                            lane = s4 * 4 + e4
                            dl = jnp.max(jnp.where(iota == lane, dls, 0))
                            e = g * 16 + lane
                            for j in range(D // 16):
                                sl = pl.ds(j * 16, 16)
                                v = gbuf[e, sl]
                                plsc.addupdate(asum.at[dl, sl], v)
                                plsc.addupdate(asq.at[dl, sl], v * v)
                                amn[dl, sl] = jnp.minimum(amn[dl, sl], v)
                                amx[dl, sl] = jnp.maximum(amx[dl, sl], v)
                            if t == 0:
                                plsc.addupdate(acnt.at[dl, pl.ds(0, 16)],
                                               ones)
                        return 0

                    lax.fori_loop(0, 4, sub4, 0)
                    return 0

                lax.fori_loop(0, CEDGE // 16, grp16, 0)
                return 0

            lax.fori_loop(0, nch, chunk_body, 0)

            rows = pl.ds(pl.multiple_of(r * RN, 8), RN)
            pltpu.sync_copy(asum.at[pl.ds(0, RN)], s1.at[rows])
            pltpu.sync_copy(asq.at[pl.ds(0, RN)], s2.at[rows])
            pltpu.sync_copy(amn.at[pl.ds(0, RN)], mno.at[rows])
            pltpu.sync_copy(amx.at[pl.ds(0, RN)], mxo.at[rows])
            if t == 0:
                pltpu.sync_copy(acnt.at[pl.ds(0, RN)], cnto.at[rows])
            return 0

        lax.fori_loop(0, RPW, range_body, 0)


BN = 1000


def _pre_tc(h, wd, ws, bias):
    def body(h_ref, wd_ref, ws_ref, b_ref, a_ref, b0_ref, b1_ref):
        hb = h_ref[...]
        a_ref[...] = (
            jnp.dot(hb, wd_ref[...], preferred_element_type=jnp.float32)
            + b_ref[...]
        )
        bb = jnp.dot(hb, ws_ref[...], preferred_element_type=jnp.float32)
        b0_ref[...] = bb[:, :D]
        b1_ref[...] = bb[:, D:]

    return pl.pallas_call(
        body,
        grid=(N // BN,),
        in_specs=[
            pl.BlockSpec((BN, D), lambda i: (i, 0)),
            pl.BlockSpec((D, DF), lambda i: (0, 0)),
            pl.BlockSpec((D, DF), lambda i: (0, 0)),
            pl.BlockSpec((1, DF), lambda i: (0, 0)),
        ],
        out_specs=[
            pl.BlockSpec((BN, DF), lambda i: (i, 0)),
            pl.BlockSpec((BN, D), lambda i: (i, 0)),
            pl.BlockSpec((BN, D), lambda i: (i, 0)),
        ],
        out_shape=[
            jax.ShapeDtypeStruct((N, DF), jnp.float32),
            jax.ShapeDtypeStruct((N, D), jnp.float32),
            jax.ShapeDtypeStruct((N, D), jnp.float32),
        ],
        compiler_params=pltpu.CompilerParams(
            dimension_semantics=("arbitrary",)),
    )(h, wd, ws, bias)


def _post_tc(h, a_cat, aggs, cnt16, px, pr, pb, lw, lb):
    def body(h_ref, a_ref, s1a_ref, s2a_ref, mna_ref, mxa_ref,
             s1b_ref, s2b_ref, mnb_ref, mxb_ref, cnt_ref,
             px_ref, pr_ref, pb_ref, lw_ref, lb_ref,
             hp_ref, ps_ref, pq_ref):
        i = pl.program_id(0)
        cnt = cnt_ref[...][:, :1]
        cnt_c = jnp.maximum(cnt, 1.0)
        lg = jnp.log(cnt_c + 1.0)
        amp = lg / AVG_LOG
        att = AVG_LOG / lg
        has = cnt > 0.0
        hb = h_ref[...]
        tower_refs = ((s1a_ref, s2a_ref, mna_ref, mxa_ref),
                      (s1b_ref, s2b_ref, mnb_ref, mxb_ref))
        outs = []
        for t in range(2):
            fsl = pl.ds(t * D, D)
            osl = pl.ds(t * F_OUT, F_OUT)
            a = a_ref[:, fsl]
            s1_ref, s2_ref, mn_ref, mx_ref = tower_refs[t]
            ss1 = s1_ref[...]
            ss2 = s2_ref[...]
            ex1 = ss1 / cnt_c
            mean = cnt * a / cnt_c + ex1
            var = jnp.where(has, jnp.maximum(ss2 / cnt_c - ex1 * ex1, 0.0),
                            0.0)
            std = jnp.sqrt(var + 1e-5)
            mn_ = jnp.where(has, a + mn_ref[...], 0.0)
            mx_ = jnp.where(has, a + mx_ref[...], 0.0)
            agg = jnp.concatenate([mean, mn_, mx_, std], axis=-1)
            scaled = jnp.concatenate([agg, agg * amp, agg * att], axis=-1)
            o = (
                jnp.dot(hb, px_ref[:, osl],
                        preferred_element_type=jnp.float32)
                + jnp.dot(scaled, pr_ref[:, osl],
                          preferred_element_type=jnp.float32)
                + pb_ref[:, osl]
            )
            outs.append(o)
        oc = jnp.concatenate(outs, axis=-1)
        hp = jnp.dot(oc, lw_ref[...],
                     preferred_element_type=jnp.float32) + lb_ref[...]
        hp_ref[...] = hp

        @pl.when(i == 0)
        def _():
            ps_ref[...] = jnp.zeros_like(ps_ref)
            pq_ref[...] = jnp.zeros_like(pq_ref)

        ps_ref[...] += jnp.broadcast_to(
            jnp.sum(hp, axis=0, keepdims=True), ps_ref.shape)
        pq_ref[...] += jnp.broadcast_to(
            jnp.sum(hp * hp, axis=0, keepdims=True), pq_ref.shape)

    return pl.pallas_call(
        body,
        grid=(N // BN,),
        in_specs=[
            pl.BlockSpec((BN, D), lambda i: (i, 0)),
            pl.BlockSpec((BN, DF), lambda i: (i, 0)),
        ] + [pl.BlockSpec((BN, D), lambda i: (i, 0))] * 8 + [
            pl.BlockSpec((BN, 16), lambda i: (i, 0)),
            pl.BlockSpec((D, 2 * F_OUT), lambda i: (0, 0)),
            pl.BlockSpec((12 * D, 2 * F_OUT), lambda i: (0, 0)),
            pl.BlockSpec((1, 2 * F_OUT), lambda i: (0, 0)),
            pl.BlockSpec((D, D), lambda i: (0, 0)),
            pl.BlockSpec((1, D), lambda i: (0, 0)),
        ],
        out_specs=[
            pl.BlockSpec((BN, D), lambda i: (i, 0)),
            pl.BlockSpec((8, D), lambda i: (0, 0)),
            pl.BlockSpec((8, D), lambda i: (0, 0)),
        ],
        out_shape=[
            jax.ShapeDtypeStruct((N, D), jnp.float32),
            jax.ShapeDtypeStruct((8, D), jnp.float32),
            jax.ShapeDtypeStruct((8, D), jnp.float32),
        ],
        compiler_params=pltpu.CompilerParams(
            dimension_semantics=("arbitrary",)),
    )(h, a_cat, *aggs, cnt16, px, pr, pb, lw, lb)


def _norm_block(hp_ref, ps_ref, pq_ref, w_ref, b_ref, ms_ref):
    s = ps_ref[0:1, :]
    q = pq_ref[0:1, :]
    mean = s / N
    mm = mean * ms_ref[...]
    var = q / N - 2.0 * mm * mean + mm * mm
    hn = w_ref[...] * (hp_ref[...] - mm) / jnp.sqrt(var + 1e-5) + b_ref[...]
    return jnp.maximum(hn, 0.0)


def _norm_pre_tc(hp, ps, pq, gw, gb, gms, wd, ws, bias):
    def body(hp_ref, ps_ref, pq_ref, w_ref, b_ref, ms_ref,
             wd_ref, ws_ref, pb_ref, hn_ref, a_ref, b0_ref, b1_ref):
        h = _norm_block(hp_ref, ps_ref, pq_ref, w_ref, b_ref, ms_ref)
        hn_ref[...] = h
        a_ref[...] = (
            jnp.dot(h, wd_ref[...], preferred_element_type=jnp.float32)
            + pb_ref[...]
        )
        bb = jnp.dot(h, ws_ref[...], preferred_element_type=jnp.float32)
        b0_ref[...] = bb[:, :D]
        b1_ref[...] = bb[:, D:]

    return pl.pallas_call(
        body,
        grid=(N // BN,),
        in_specs=[
            pl.BlockSpec((BN, D), lambda i: (i, 0)),
            pl.BlockSpec((8, D), lambda i: (0, 0)),
            pl.BlockSpec((8, D), lambda i: (0, 0)),
            pl.BlockSpec((1, D), lambda i: (0, 0)),
            pl.BlockSpec((1, D), lambda i: (0, 0)),
            pl.BlockSpec((1, D), lambda i: (0, 0)),
            pl.BlockSpec((D, DF), lambda i: (0, 0)),
            pl.BlockSpec((D, DF), lambda i: (0, 0)),
            pl.BlockSpec((1, DF), lambda i: (0, 0)),
        ],
        out_specs=[
            pl.BlockSpec((BN, D), lambda i: (i, 0)),
            pl.BlockSpec((BN, DF), lambda i: (i, 0)),
            pl.BlockSpec((BN, D), lambda i: (i, 0)),
            pl.BlockSpec((BN, D), lambda i: (i, 0)),
        ],
        out_shape=[
            jax.ShapeDtypeStruct((N, D), jnp.float32),
            jax.ShapeDtypeStruct((N, DF), jnp.float32),
            jax.ShapeDtypeStruct((N, D), jnp.float32),
            jax.ShapeDtypeStruct((N, D), jnp.float32),
        ],
        compiler_params=pltpu.CompilerParams(
            dimension_semantics=("arbitrary",)),
    )(hp, ps, pq, gw, gb, gms, wd, ws, bias)


def _norm_fc_tc(hp, ps, pq, gw, gb, gms, w1, b1, w2, b2):
    def body(hp_ref, ps_ref, pq_ref, w_ref, b_ref, ms_ref,
             w1_ref, b1_ref, w2_ref, b2_ref, o_ref):
        h = _norm_block(hp_ref, ps_ref, pq_ref, w_ref, b_ref, ms_ref)
        h = jnp.maximum(
            jnp.dot(h, w1_ref[...], preferred_element_type=jnp.float32)
            + b1_ref[...], 0.0)
        o_ref[...] = (
            jnp.dot(h, w2_ref[...], preferred_element_type=jnp.float32)
            + b2_ref[...]
        )

    return pl.pallas_call(
        body,
        grid=(N // BN,),
        in_specs=[
            pl.BlockSpec((BN, D), lambda i: (i, 0)),
            pl.BlockSpec((8, D), lambda i: (0, 0)),
            pl.BlockSpec((8, D), lambda i: (0, 0)),
            pl.BlockSpec((1, D), lambda i: (0, 0)),
            pl.BlockSpec((1, D), lambda i: (0, 0)),
            pl.BlockSpec((1, D), lambda i: (0, 0)),
            pl.BlockSpec((D, D), lambda i: (0, 0)),
            pl.BlockSpec((1, D), lambda i: (0, 0)),
            pl.BlockSpec((D, D), lambda i: (0, 0)),
            pl.BlockSpec((1, D), lambda i: (0, 0)),
        ],
        out_specs=pl.BlockSpec((BN, D), lambda i: (i, 0)),
        out_shape=jax.ShapeDtypeStruct((N, D), jnp.float32),
        compiler_params=pltpu.CompilerParams(
            dimension_semantics=("arbitrary",)),
    )(hp, ps, pq, gw, gb, gms, w1, b1, w2, b2)


def kernel(x, edge_index, params):
    p = params
    src = edge_index[0].astype(jnp.int32)
    dst = edge_index[1].astype(jnp.int32)

    srcl, dll, cnts = _build_lists(dst, src)

    def pre_w(l):
        wd = jnp.concatenate(
            [p[f"pre_W_{l}_{t}"][:D] for t in range(2)], axis=1)
        ws = jnp.concatenate(
            [p[f"pre_W_{l}_{t}"][D:] for t in range(2)], axis=1)
        pb = jnp.concatenate(
            [p[f"pre_b_{l}_{t}"] for t in range(2)]).reshape(1, DF)
        return wd, ws, pb

    h = x
    out = None
    cnt16 = None
    for l in range(2):
        if l == 0:
            wd, ws, pb = pre_w(0)
            a_cat, b0, b1 = _pre_tc(h, wd, ws, pb)
        *aggs, cnt_new = _aggregate(b0, b1, srcl, dll, cnts)
        if cnt16 is None:
            cnt16 = cnt_new
        px = jnp.concatenate(
            [p[f"post_W_{l}_{t}"][:D] for t in range(2)], axis=1)
        pr = jnp.concatenate(
            [p[f"post_W_{l}_{t}"][D:] for t in range(2)], axis=1)
        pbo = jnp.concatenate(
            [p[f"post_b_{l}_{t}"] for t in range(2)]).reshape(1, 2 * F_OUT)
        hp, ps, pq = _post_tc(
            h, a_cat, aggs, cnt16, px, pr, pbo,
            p[f"lin_W_{l}"], p[f"lin_b_{l}"].reshape(1, D))
        gw = p[f"gn_w_{l}"].reshape(1, D)
        gb = p[f"gn_b_{l}"].reshape(1, D)
        gms = p[f"gn_ms_{l}"].reshape(1, D)
        if l == 0:
            wd, ws, pb = pre_w(1)
            h, a_cat, b0, b1 = _norm_pre_tc(hp, ps, pq, gw, gb, gms,
                                            wd, ws, pb)
        else:
            out = _norm_fc_tc(hp, ps, pq, gw, gb, gms,
                              p["fc1_W"], p["fc1_b"].reshape(1, D),
                              p["fc2_W"], p["fc2_b"].reshape(1, D))
    return out


# speedup vs baseline: 4.4763x; 1.1935x over previous
import functools

import jax
import jax.numpy as jnp
import numpy as np
from jax import lax
from jax.experimental import pallas as pl
from jax.experimental.pallas import tpu as pltpu
from jax.experimental.pallas import tpu_sc as plsc

N = 10000
E = 320000
D = 128
F_OUT = 64
DF = 2 * D
AVG_LOG = float(np.log(33.0))

NC = 2
NS = 16
NW = NC * NS
RPW = 4
NR = NW * RPW
RN = 80
NPAD = NR * RN
ACC_ROWS = RN + 1

CEDGE = 128
SUPC = 32
SUP = SUPC * CEDGE
FLUSH = 4096
SCH = 8000
GRP = 25
NBUF = FLUSH + 512
CAP = 80 * FLUSH

_SC_PARAMS = pltpu.CompilerParams(needs_layout_passes=False)


def _sc_mesh():
    return plsc.VectorSubcoreMesh(
        core_axis_name="c", subcore_axis_name="s", num_cores=NC, num_subcores=NS
    )


@functools.partial(
    pl.kernel,
    out_type=(
        jax.ShapeDtypeStruct((NR * CAP,), jnp.int32),
        jax.ShapeDtypeStruct((NR * CAP,), jnp.int32),
        jax.ShapeDtypeStruct((NR * 16,), jnp.int32),
    ),
    mesh=_sc_mesh(),
    compiler_params=_SC_PARAMS,
    scratch_types=[
        pltpu.VMEM((SCH,), jnp.int32),
        pltpu.VMEM((SCH,), jnp.int32),
    ] + [pltpu.VMEM((NBUF,), jnp.int32) for _ in range(2 * RPW)] + [
        pltpu.VMEM((16,), jnp.int32),
    ],
)
def _build_lists(dst_hbm, src_hbm, srcl_hbm, dll_hbm, cnt_hbm,
                 std, sts, bs0, bs1, bs2, bs3, bd0, bd1, bd2, bd3, cvec):
    bs = [bs0, bs1, bs2, bs3]
    bd = [bd0, bd1, bd2, bd3]
    wid = lax.axis_index("s") * NC + lax.axis_index("c")
    wbase = wid * (RPW * RN)
    iota = lax.iota(jnp.int32, 16)
    dummy_dl = jnp.full((16,), RN, jnp.int32)

    def chunk_body(ch, carry):
        pltpu.sync_copy(dst_hbm.at[pl.ds(ch * SCH, SCH)], std)
        pltpu.sync_copy(src_hbm.at[pl.ds(ch * SCH, SCH)], sts)

        def grp_body(g, c2):
            offs = list(c2[:RPW])
            goffs = list(c2[RPW:])
            for u in range(GRP):
                sl = pl.ds((g * GRP + u) * 16, 16)
                dv = std[sl]
                sv = sts[sl]
                q = dv - wbase
                for kk in range(RPW):
                    m = (q >= kk * RN) & (q < (kk + 1) * RN)
                    m32 = jnp.where(m, jnp.int32(1), jnp.int32(0))
                    pc = plsc.cumsum(m32)
                    popc = plsc.all_reduce_population_count(m)
                    tgt = offs[kk] + pc - m32
                    plsc.store_scatter(bs[kk], [tgt], sv, mask=m)
                    plsc.store_scatter(bd[kk], [tgt], q - kk * RN, mask=m)
                    offs[kk] = offs[kk] + popc
            for kk in range(RPW):
                r = wid * RPW + kk
                off_s = jnp.max(offs[kk])

                def do_flush(op, kk=kk, r=r):
                    off_v, go = op
                    base = pl.multiple_of(r * CAP + go, 8)
                    pltpu.sync_copy(bs[kk].at[pl.ds(0, FLUSH)],
                                    srcl_hbm.at[pl.ds(base, FLUSH)])
                    pltpu.sync_copy(bd[kk].at[pl.ds(0, FLUSH)],
                                    dll_hbm.at[pl.ds(base, FLUSH)])
                    for t in range((NBUF - FLUSH) // 16):
                        s_src = pl.ds(FLUSH + t * 16, 16)
                        s_dst = pl.ds(t * 16, 16)
                        bs[kk][s_dst] = bs[kk][s_src]
                        bd[kk][s_dst] = bd[kk][s_src]
                    return (off_v - FLUSH, go + FLUSH)

                offs[kk], goffs[kk] = lax.cond(
                    off_s >= FLUSH, do_flush, lambda op: op,
                    (offs[kk], goffs[kk]))
            return tuple(offs) + tuple(goffs)

        return lax.fori_loop(0, SCH // 16 // GRP, grp_body, carry)

    zero_v = jnp.zeros((16,), jnp.int32)
    zero_s = jnp.int32(0)
    carry = (zero_v,) * RPW + (zero_s,) * RPW
    carry = lax.fori_loop(0, E // SCH, chunk_body, carry)

    for kk in range(RPW):
        r = wid * RPW + kk
        off_s = jnp.max(carry[kk])
        go = carry[RPW + kk]
        start = (off_s // 16) * 16
        pos = start + iota
        plsc.store_scatter(bs[kk], [pos], iota, mask=pos >= off_s)
        plsc.store_scatter(bd[kk], [pos], dummy_dl, mask=pos >= off_s)
        for t in range(1, 9):
            bs[kk][pl.ds(start + t * 16, 16)] = iota
            bd[kk][pl.ds(start + t * 16, 16)] = dummy_dl
        base = pl.multiple_of(r * CAP + go, 8)
        pltpu.sync_copy(bs[kk].at[pl.ds(0, FLUSH)],
                        srcl_hbm.at[pl.ds(base, FLUSH)])
        pltpu.sync_copy(bd[kk].at[pl.ds(0, FLUSH)],
                        dll_hbm.at[pl.ds(base, FLUSH)])
        cvec[...] = jnp.full((16,), go + off_s, jnp.int32)
        pltpu.sync_copy(cvec, cnt_hbm.at[pl.ds(pl.multiple_of(r * 16, 8), 16)])


@functools.partial(
    pl.kernel,
    out_type=tuple(
        [jax.ShapeDtypeStruct((NPAD, D), jnp.float32)] * 8
        + [jax.ShapeDtypeStruct((NPAD, 16), jnp.float32)]
    ),
    mesh=_sc_mesh(),
    compiler_params=_SC_PARAMS,
    scratch_types=[
        pltpu.VMEM((ACC_ROWS, D), jnp.float32),
        pltpu.VMEM((ACC_ROWS, D), jnp.float32),
        pltpu.VMEM((ACC_ROWS, D), jnp.float32),
        pltpu.VMEM((ACC_ROWS, D), jnp.float32),
        pltpu.VMEM((ACC_ROWS, 16), jnp.float32),
        pltpu.VMEM((CEDGE, D), jnp.float32),
        pltpu.VMEM((CEDGE, D), jnp.float32),
        pltpu.VMEM((SUP,), jnp.int32),
        pltpu.VMEM((SUP,), jnp.int32),
        pltpu.VMEM((CEDGE,), jnp.int32),
        pltpu.VMEM((CEDGE,), jnp.int32),
        pltpu.VMEM((16,), jnp.int32),
        pltpu.SemaphoreType.DMA,
        pltpu.SemaphoreType.DMA,
    ],
)
def _aggregate(tab0, tab1, srcl_hbm, dll_hbm, cnt_hbm,
               s1a, s2a, mna, mxa, s1b, s2b, mnb, mxb, cnto,
               asum, asq, amn, amx, acnt, gbufA, gbufB, sidx_st, dlv_st,
               dlcA, dlcB, cvec, semA, semB):
    wid = lax.axis_index("s") * NC + lax.axis_index("c")
    iota = lax.iota(jnp.int32, 16)
    zero16 = jnp.zeros((16,), jnp.float32)
    pinf = jnp.full((16,), jnp.inf, jnp.float32)
    ninf = jnp.full((16,), -jnp.inf, jnp.float32)
    ones = jnp.ones((16,), jnp.float32)

    for t, (tab, s1, s2, mno, mxo) in enumerate(
        ((tab0, s1a, s2a, mna, mxa), (tab1, s1b, s2b, mnb, mxb))
    ):
        def range_body(kk, _0, tab=tab, s1=s1, s2=s2, mno=mno, mxo=mxo, t=t):
            r = wid * RPW + kk

            def init_row(i, _):
                for j in range(D // 16):
                    sl = pl.ds(j * 16, 16)
                    asum[i, sl] = zero16
                    asq[i, sl] = zero16
                    amn[i, sl] = pinf
                    amx[i, sl] = ninf
                acnt[i, pl.ds(0, 16)] = zero16
                return 0

            lax.fori_loop(0, ACC_ROWS, init_row, 0)

            pltpu.sync_copy(
                cnt_hbm.at[pl.ds(pl.multiple_of(r * 16, 8), 16)], cvec)
            cn = jnp.max(cvec[...])
            nch = (cn + CEDGE - 1) // CEDGE

            def prep(c, dlc, tab=tab):
                sup = c // SUPC
                local = c - sup * SUPC

                @pl.when(local == 0)
                def _():
                    base = pl.multiple_of(r * CAP + sup * SUP, 8)
                    pltpu.sync_copy(srcl_hbm.at[pl.ds(base, SUP)], sidx_st)
                    pltpu.sync_copy(dll_hbm.at[pl.ds(base, SUP)], dlv_st)

                off = local * CEDGE
                for v16 in range(CEDGE // 16):
                    sl = pl.ds(v16 * 16, 16)
                    dlc[sl] = dlv_st[pl.ds(off + v16 * 16, 16)]
                return sidx_st.at[pl.ds(off, CEDGE)]

            def start_gather(c, gbuf, sem, dlc, tab=tab):
                idx = prep(c, dlc)
                return pltpu.async_copy(tab.at[idx], gbuf, sem)

            def compute(c, gbuf, dlc, t=t):
                def grp16(g, _2):
                    dls = dlc[pl.ds(g * 16, 16)]

                    def sub4(s4, _3):
                        for e4 in range(4):
                            lane = s4 * 4 + e4
                            dl = jnp.max(jnp.where(iota == lane, dls, 0))
                            e = g * 16 + lane
                            for j in range(D // 16):
                                sl = pl.ds(j * 16, 16)
                                v = gbuf[e, sl]
                                plsc.addupdate(asum.at[dl, sl], v)
                                plsc.addupdate(asq.at[dl, sl], v * v)
                                amn[dl, sl] = jnp.minimum(amn[dl, sl], v)
                                amx[dl, sl] = jnp.maximum(amx[dl, sl], v)
                            if t == 0:
                                plsc.addupdate(acnt.at[dl, pl.ds(0, 16)],
                                               ones)
                        return 0

                    lax.fori_loop(0, 4, sub4, 0)
                    return 0

                lax.fori_loop(0, CEDGE // 16, grp16, 0)

            @pl.when(nch > 0)
            def _():
                start_gather(0, gbufA, semA, dlcA)

            def pair_body(i, _):
                c0 = 2 * i
                c1 = c0 + 1
                pltpu.make_async_copy(tab.at[sidx_st.at[pl.ds(0, CEDGE)]],
                                      gbufA, semA).wait()

                @pl.when(c1 < nch)
                def _():
                    start_gather(c1, gbufB, semB, dlcB)

                compute(c0, gbufA, dlcA)

                @pl.when(c1 < nch)
                def _():
                    pltpu.make_async_copy(
                        tab.at[sidx_st.at[pl.ds(0, CEDGE)]],
                        gbufB, semB).wait()

                    @pl.when(c1 + 1 < nch)
                    def _():
                        start_gather(c1 + 1, gbufA, semA, dlcA)

                    compute(c1, gbufB, dlcB)
                return 0

            lax.fori_loop(0, (nch + 1) // 2, pair_body, 0)

            rows = pl.ds(pl.multiple_of(r * RN, 8), RN)
            pltpu.sync_copy(asum.at[pl.ds(0, RN)], s1.at[rows])
            pltpu.sync_copy(asq.at[pl.ds(0, RN)], s2.at[rows])
            pltpu.sync_copy(amn.at[pl.ds(0, RN)], mno.at[rows])
            pltpu.sync_copy(amx.at[pl.ds(0, RN)], mxo.at[rows])
            if t == 0:
                pltpu.sync_copy(acnt.at[pl.ds(0, RN)], cnto.at[rows])
            return 0

        lax.fori_loop(0, RPW, range_body, 0)


BN = 1000


def _pre_tc(h, wd, ws, bias):
    def body(h_ref, wd_ref, ws_ref, b_ref, a_ref, b0_ref, b1_ref):
        hb = h_ref[...]
        a_ref[...] = (
            jnp.dot(hb, wd_ref[...], preferred_element_type=jnp.float32)
            + b_ref[...]
        )
        bb = jnp.dot(hb, ws_ref[...], preferred_element_type=jnp.float32)
        b0_ref[...] = bb[:, :D]
        b1_ref[...] = bb[:, D:]

    return pl.pallas_call(
        body,
        grid=(N // BN,),
        in_specs=[
            pl.BlockSpec((BN, D), lambda i: (i, 0)),
            pl.BlockSpec((D, DF), lambda i: (0, 0)),
            pl.BlockSpec((D, DF), lambda i: (0, 0)),
            pl.BlockSpec((1, DF), lambda i: (0, 0)),
        ],
        out_specs=[
            pl.BlockSpec((BN, DF), lambda i: (i, 0)),
            pl.BlockSpec((BN, D), lambda i: (i, 0)),
            pl.BlockSpec((BN, D), lambda i: (i, 0)),
        ],
        out_shape=[
            jax.ShapeDtypeStruct((N, DF), jnp.float32),
            jax.ShapeDtypeStruct((N, D), jnp.float32),
            jax.ShapeDtypeStruct((N, D), jnp.float32),
        ],
        compiler_params=pltpu.CompilerParams(
            dimension_semantics=("arbitrary",)),
    )(h, wd, ws, bias)


def _post_tc(h, a_cat, aggs, cnt16, px, pr, pb, lw, lb):
    def body(h_ref, a_ref, s1a_ref, s2a_ref, mna_ref, mxa_ref,
             s1b_ref, s2b_ref, mnb_ref, mxb_ref, cnt_ref,
             px_ref, pr_ref, pb_ref, lw_ref, lb_ref,
             hp_ref, ps_ref, pq_ref):
        i = pl.program_id(0)
        cnt = cnt_ref[...][:, :1]
        cnt_c = jnp.maximum(cnt, 1.0)
        lg = jnp.log(cnt_c + 1.0)
        amp = lg / AVG_LOG
        att = AVG_LOG / lg
        has = cnt > 0.0
        hb = h_ref[...]
        tower_refs = ((s1a_ref, s2a_ref, mna_ref, mxa_ref),
                      (s1b_ref, s2b_ref, mnb_ref, mxb_ref))
        outs = []
        for t in range(2):
            fsl = pl.ds(t * D, D)
            osl = pl.ds(t * F_OUT, F_OUT)
            a = a_ref[:, fsl]
            s1_ref, s2_ref, mn_ref, mx_ref = tower_refs[t]
            ss1 = s1_ref[...]
            ss2 = s2_ref[...]
            ex1 = ss1 / cnt_c
            mean = cnt * a / cnt_c + ex1
            var = jnp.where(has, jnp.maximum(ss2 / cnt_c - ex1 * ex1, 0.0),
                            0.0)
            std = jnp.sqrt(var + 1e-5)
            mn_ = jnp.where(has, a + mn_ref[...], 0.0)
            mx_ = jnp.where(has, a + mx_ref[...], 0.0)
            agg = jnp.concatenate([mean, mn_, mx_, std], axis=-1)
            scaled = jnp.concatenate([agg, agg * amp, agg * att], axis=-1)
            o = (
                jnp.dot(hb, px_ref[:, osl],
                        preferred_element_type=jnp.float32)
                + jnp.dot(scaled, pr_ref[:, osl],
                          preferred_element_type=jnp.float32)
                + pb_ref[:, osl]
            )
            outs.append(o)
        oc = jnp.concatenate(outs, axis=-1)
        hp = jnp.dot(oc, lw_ref[...],
                     preferred_element_type=jnp.float32) + lb_ref[...]
        hp_ref[...] = hp

        @pl.when(i == 0)
        def _():
            ps_ref[...] = jnp.zeros_like(ps_ref)
            pq_ref[...] = jnp.zeros_like(pq_ref)

        ps_ref[...] += jnp.broadcast_to(
            jnp.sum(hp, axis=0, keepdims=True), ps_ref.shape)
        pq_ref[...] += jnp.broadcast_to(
            jnp.sum(hp * hp, axis=0, keepdims=True), pq_ref.shape)

    return pl.pallas_call(
        body,
        grid=(N // BN,),
        in_specs=[
            pl.BlockSpec((BN, D), lambda i: (i, 0)),
            pl.BlockSpec((BN, DF), lambda i: (i, 0)),
        ] + [pl.BlockSpec((BN, D), lambda i: (i, 0))] * 8 + [
            pl.BlockSpec((BN, 16), lambda i: (i, 0)),
            pl.BlockSpec((D, 2 * F_OUT), lambda i: (0, 0)),
            pl.BlockSpec((12 * D, 2 * F_OUT), lambda i: (0, 0)),
            pl.BlockSpec((1, 2 * F_OUT), lambda i: (0, 0)),
            pl.BlockSpec((D, D), lambda i: (0, 0)),
            pl.BlockSpec((1, D), lambda i: (0, 0)),
        ],
        out_specs=[
            pl.BlockSpec((BN, D), lambda i: (i, 0)),
            pl.BlockSpec((8, D), lambda i: (0, 0)),
            pl.BlockSpec((8, D), lambda i: (0, 0)),
        ],
        out_shape=[
            jax.ShapeDtypeStruct((N, D), jnp.float32),
            jax.ShapeDtypeStruct((8, D), jnp.float32),
            jax.ShapeDtypeStruct((8, D), jnp.float32),
        ],
        compiler_params=pltpu.CompilerParams(
            dimension_semantics=("arbitrary",)),
    )(h, a_cat, *aggs, cnt16, px, pr, pb, lw, lb)


def _norm_block(hp_ref, ps_ref, pq_ref, w_ref, b_ref, ms_ref):
    s = ps_ref[0:1, :]
    q = pq_ref[0:1, :]
    mean = s / N
    mm = mean * ms_ref[...]
    var = q / N - 2.0 * mm * mean + mm * mm
    hn = w_ref[...] * (hp_ref[...] - mm) / jnp.sqrt(var + 1e-5) + b_ref[...]
    return jnp.maximum(hn, 0.0)


def _norm_pre_tc(hp, ps, pq, gw, gb, gms, wd, ws, bias):
    def body(hp_ref, ps_ref, pq_ref, w_ref, b_ref, ms_ref,
             wd_ref, ws_ref, pb_ref, hn_ref, a_ref, b0_ref, b1_ref):
        h = _norm_block(hp_ref, ps_ref, pq_ref, w_ref, b_ref, ms_ref)
        hn_ref[...] = h
        a_ref[...] = (
            jnp.dot(h, wd_ref[...], preferred_element_type=jnp.float32)
            + pb_ref[...]
        )
        bb = jnp.dot(h, ws_ref[...], preferred_element_type=jnp.float32)
        b0_ref[...] = bb[:, :D]
        b1_ref[...] = bb[:, D:]

    return pl.pallas_call(
        body,
        grid=(N // BN,),
        in_specs=[
            pl.BlockSpec((BN, D), lambda i: (i, 0)),
            pl.BlockSpec((8, D), lambda i: (0, 0)),
            pl.BlockSpec((8, D), lambda i: (0, 0)),
            pl.BlockSpec((1, D), lambda i: (0, 0)),
            pl.BlockSpec((1, D), lambda i: (0, 0)),
            pl.BlockSpec((1, D), lambda i: (0, 0)),
            pl.BlockSpec((D, DF), lambda i: (0, 0)),
            pl.BlockSpec((D, DF), lambda i: (0, 0)),
            pl.BlockSpec((1, DF), lambda i: (0, 0)),
        ],
        out_specs=[
            pl.BlockSpec((BN, D), lambda i: (i, 0)),
            pl.BlockSpec((BN, DF), lambda i: (i, 0)),
            pl.BlockSpec((BN, D), lambda i: (i, 0)),
            pl.BlockSpec((BN, D), lambda i: (i, 0)),
        ],
        out_shape=[
            jax.ShapeDtypeStruct((N, D), jnp.float32),
            jax.ShapeDtypeStruct((N, DF), jnp.float32),
            jax.ShapeDtypeStruct((N, D), jnp.float32),
            jax.ShapeDtypeStruct((N, D), jnp.float32),
        ],
        compiler_params=pltpu.CompilerParams(
            dimension_semantics=("arbitrary",)),
    )(hp, ps, pq, gw, gb, gms, wd, ws, bias)


def _norm_fc_tc(hp, ps, pq, gw, gb, gms, w1, b1, w2, b2):
    def body(hp_ref, ps_ref, pq_ref, w_ref, b_ref, ms_ref,
             w1_ref, b1_ref, w2_ref, b2_ref, o_ref):
        h = _norm_block(hp_ref, ps_ref, pq_ref, w_ref, b_ref, ms_ref)
        h = jnp.maximum(
            jnp.dot(h, w1_ref[...], preferred_element_type=jnp.float32)
            + b1_ref[...], 0.0)
        o_ref[...] = (
            jnp.dot(h, w2_ref[...], preferred_element_type=jnp.float32)
            + b2_ref[...]
        )

    return pl.pallas_call(
        body,
        grid=(N // BN,),
        in_specs=[
            pl.BlockSpec((BN, D), lambda i: (i, 0)),
            pl.BlockSpec((8, D), lambda i: (0, 0)),
            pl.BlockSpec((8, D), lambda i: (0, 0)),
            pl.BlockSpec((1, D), lambda i: (0, 0)),
            pl.BlockSpec((1, D), lambda i: (0, 0)),
            pl.BlockSpec((1, D), lambda i: (0, 0)),
            pl.BlockSpec((D, D), lambda i: (0, 0)),
            pl.BlockSpec((1, D), lambda i: (0, 0)),
            pl.BlockSpec((D, D), lambda i: (0, 0)),
            pl.BlockSpec((1, D), lambda i: (0, 0)),
        ],
        out_specs=pl.BlockSpec((BN, D), lambda i: (i, 0)),
        out_shape=jax.ShapeDtypeStruct((N, D), jnp.float32),
        compiler_params=pltpu.CompilerParams(
            dimension_semantics=("arbitrary",)),
    )(hp, ps, pq, gw, gb, gms, w1, b1, w2, b2)


def kernel(x, edge_index, params):
    p = params
    src = edge_index[0].astype(jnp.int32)
    dst = edge_index[1].astype(jnp.int32)

    srcl, dll, cnts = _build_lists(dst, src)

    def pre_w(l):
        wd = jnp.concatenate(
            [p[f"pre_W_{l}_{t}"][:D] for t in range(2)], axis=1)
        ws = jnp.concatenate(
            [p[f"pre_W_{l}_{t}"][D:] for t in range(2)], axis=1)
        pb = jnp.concatenate(
            [p[f"pre_b_{l}_{t}"] for t in range(2)]).reshape(1, DF)
        return wd, ws, pb

    h = x
    out = None
    cnt16 = None
    for l in range(2):
        if l == 0:
            wd, ws, pb = pre_w(0)
            a_cat, b0, b1 = _pre_tc(h, wd, ws, pb)
        *aggs, cnt_new = _aggregate(b0, b1, srcl, dll, cnts)
        if cnt16 is None:
            cnt16 = cnt_new
        px = jnp.concatenate(
            [p[f"post_W_{l}_{t}"][:D] for t in range(2)], axis=1)
        pr = jnp.concatenate(
            [p[f"post_W_{l}_{t}"][D:] for t in range(2)], axis=1)
        pbo = jnp.concatenate(
            [p[f"post_b_{l}_{t}"] for t in range(2)]).reshape(1, 2 * F_OUT)
        hp, ps, pq = _post_tc(
            h, a_cat, aggs, cnt16, px, pr, pbo,
            p[f"lin_W_{l}"], p[f"lin_b_{l}"].reshape(1, D))
        gw = p[f"gn_w_{l}"].reshape(1, D)
        gb = p[f"gn_b_{l}"].reshape(1, D)
        gms = p[f"gn_ms_{l}"].reshape(1, D)
        if l == 0:
            wd, ws, pb = pre_w(1)
            h, a_cat, b0, b1 = _norm_pre_tc(hp, ps, pq, gw, gb, gms,
                                            wd, ws, pb)
        else:
            out = _norm_fc_tc(hp, ps, pq, gw, gb, gms,
                              p["fc1_W"], p["fc1_b"].reshape(1, D),
                              p["fc2_W"], p["fc2_b"].reshape(1, D))
    return out


# speedup vs baseline: 4.5711x; 1.0212x over previous
import functools

import jax
import jax.numpy as jnp
import numpy as np
from jax import lax
from jax.experimental import pallas as pl
from jax.experimental.pallas import tpu as pltpu
from jax.experimental.pallas import tpu_sc as plsc

N = 10000
E = 320000
D = 128
F_OUT = 64
DF = 2 * D
AVG_LOG = float(np.log(33.0))

NC = 2
NS = 16
NW = NC * NS
RPW = 4
NR = NW * RPW
RN = 80
NPAD = NR * RN
ACC_ROWS = RN + 1

CEDGE = 128
SUPC = 32
SUP = SUPC * CEDGE
FLUSH = 4096
SCH = 8000
GRP = 25
NBUF = FLUSH + 512
CAP = 80 * FLUSH

_SC_PARAMS = pltpu.CompilerParams(needs_layout_passes=False)


def _sc_mesh():
    return plsc.VectorSubcoreMesh(
        core_axis_name="c", subcore_axis_name="s", num_cores=NC, num_subcores=NS
    )


@functools.partial(
    pl.kernel,
    out_type=(
        jax.ShapeDtypeStruct((NR * CAP,), jnp.int32),
        jax.ShapeDtypeStruct((NR * CAP,), jnp.int32),
        jax.ShapeDtypeStruct((NR * 16,), jnp.int32),
    ),
    mesh=_sc_mesh(),
    compiler_params=_SC_PARAMS,
    scratch_types=[
        pltpu.VMEM((SCH,), jnp.int32),
        pltpu.VMEM((SCH,), jnp.int32),
    ] + [pltpu.VMEM((NBUF,), jnp.int32) for _ in range(2 * RPW)] + [
        pltpu.VMEM((16,), jnp.int32),
    ],
)
def _build_lists(dst_hbm, src_hbm, srcl_hbm, dll_hbm, cnt_hbm,
                 std, sts, bs0, bs1, bs2, bs3, bd0, bd1, bd2, bd3, cvec):
    bs = [bs0, bs1, bs2, bs3]
    bd = [bd0, bd1, bd2, bd3]
    wid = lax.axis_index("s") * NC + lax.axis_index("c")
    wbase = wid * (RPW * RN)
    iota = lax.iota(jnp.int32, 16)
    dummy_dl = jnp.full((16,), RN, jnp.int32)

    def chunk_body(ch, carry):
        pltpu.sync_copy(dst_hbm.at[pl.ds(ch * SCH, SCH)], std)
        pltpu.sync_copy(src_hbm.at[pl.ds(ch * SCH, SCH)], sts)

        def grp_body(g, c2):
            offs = list(c2[:RPW])
            goffs = list(c2[RPW:])
            for u in range(GRP):
                sl = pl.ds((g * GRP + u) * 16, 16)
                dv = std[sl]
                sv = sts[sl]
                q = dv - wbase
                for kk in range(RPW):
                    m = (q >= kk * RN) & (q < (kk + 1) * RN)
                    m32 = jnp.where(m, jnp.int32(1), jnp.int32(0))
                    pc = plsc.cumsum(m32)
                    popc = plsc.all_reduce_population_count(m)
                    tgt = offs[kk] + pc - m32
                    plsc.store_scatter(bs[kk], [tgt], sv, mask=m)
                    plsc.store_scatter(bd[kk], [tgt], q - kk * RN, mask=m)
                    offs[kk] = offs[kk] + popc
            for kk in range(RPW):
                r = wid * RPW + kk
                off_s = jnp.max(offs[kk])

                def do_flush(op, kk=kk, r=r):
                    off_v, go = op
                    base = pl.multiple_of(r * CAP + go, 8)
                    pltpu.sync_copy(bs[kk].at[pl.ds(0, FLUSH)],
                                    srcl_hbm.at[pl.ds(base, FLUSH)])
                    pltpu.sync_copy(bd[kk].at[pl.ds(0, FLUSH)],
                                    dll_hbm.at[pl.ds(base, FLUSH)])
                    for t in range((NBUF - FLUSH) // 16):
                        s_src = pl.ds(FLUSH + t * 16, 16)
                        s_dst = pl.ds(t * 16, 16)
                        bs[kk][s_dst] = bs[kk][s_src]
                        bd[kk][s_dst] = bd[kk][s_src]
                    return (off_v - FLUSH, go + FLUSH)

                offs[kk], goffs[kk] = lax.cond(
                    off_s >= FLUSH, do_flush, lambda op: op,
                    (offs[kk], goffs[kk]))
            return tuple(offs) + tuple(goffs)

        return lax.fori_loop(0, SCH // 16 // GRP, grp_body, carry)

    zero_v = jnp.zeros((16,), jnp.int32)
    zero_s = jnp.int32(0)
    carry = (zero_v,) * RPW + (zero_s,) * RPW
    carry = lax.fori_loop(0, E // SCH, chunk_body, carry)

    for kk in range(RPW):
        r = wid * RPW + kk
        off_s = jnp.max(carry[kk])
        go = carry[RPW + kk]
        start = (off_s // 16) * 16
        pos = start + iota
        plsc.store_scatter(bs[kk], [pos], iota, mask=pos >= off_s)
        plsc.store_scatter(bd[kk], [pos], dummy_dl, mask=pos >= off_s)
        for t in range(1, 9):
            bs[kk][pl.ds(start + t * 16, 16)] = iota
            bd[kk][pl.ds(start + t * 16, 16)] = dummy_dl
        base = pl.multiple_of(r * CAP + go, 8)
        pltpu.sync_copy(bs[kk].at[pl.ds(0, FLUSH)],
                        srcl_hbm.at[pl.ds(base, FLUSH)])
        pltpu.sync_copy(bd[kk].at[pl.ds(0, FLUSH)],
                        dll_hbm.at[pl.ds(base, FLUSH)])
        cvec[...] = jnp.full((16,), go + off_s, jnp.int32)
        pltpu.sync_copy(cvec, cnt_hbm.at[pl.ds(pl.multiple_of(r * 16, 8), 16)])


@functools.partial(
    pl.kernel,
    out_type=tuple(
        [jax.ShapeDtypeStruct((NPAD, D), jnp.float32)] * 8
        + [jax.ShapeDtypeStruct((NPAD, 16), jnp.float32)]
    ),
    mesh=_sc_mesh(),
    compiler_params=_SC_PARAMS,
    scratch_types=[
        pltpu.VMEM((ACC_ROWS, D), jnp.float32),
        pltpu.VMEM((ACC_ROWS, D), jnp.float32),
        pltpu.VMEM((ACC_ROWS, D), jnp.float32),
        pltpu.VMEM((ACC_ROWS, D), jnp.float32),
        pltpu.VMEM((ACC_ROWS, 16), jnp.float32),
        pltpu.VMEM((CEDGE, D), jnp.float32),
        pltpu.VMEM((CEDGE, D), jnp.float32),
        pltpu.VMEM((SUP,), jnp.int32),
        pltpu.VMEM((SUP,), jnp.int32),
        pltpu.VMEM((CEDGE,), jnp.int32),
        pltpu.VMEM((CEDGE,), jnp.int32),
        pltpu.VMEM((16,), jnp.int32),
        pltpu.SemaphoreType.DMA,
        pltpu.SemaphoreType.DMA,
    ],
)
def _aggregate(tab0, tab1, srcl_hbm, dll_hbm, cnt_hbm,
               s1a, s2a, mna, mxa, s1b, s2b, mnb, mxb, cnto,
               asum, asq, amn, amx, acnt, gbufA, gbufB, sidx_st, dlv_st,
               dlcA, dlcB, cvec, semA, semB):
    wid = lax.axis_index("s") * NC + lax.axis_index("c")
    iota = lax.iota(jnp.int32, 16)
    zero16 = jnp.zeros((16,), jnp.float32)
    pinf = jnp.full((16,), jnp.inf, jnp.float32)
    ninf = jnp.full((16,), -jnp.inf, jnp.float32)
    ones = jnp.ones((16,), jnp.float32)

    for t, (tab, s1, s2, mno, mxo) in enumerate(
        ((tab0, s1a, s2a, mna, mxa), (tab1, s1b, s2b, mnb, mxb))
    ):
        def range_body(kk, _0, tab=tab, s1=s1, s2=s2, mno=mno, mxo=mxo, t=t):
            r = wid * RPW + kk

            def init_row(i, _):
                for j in range(D // 16):
                    sl = pl.ds(j * 16, 16)
                    asum[i, sl] = zero16
                    asq[i, sl] = zero16
                    amn[i, sl] = pinf
                    amx[i, sl] = ninf
                acnt[i, pl.ds(0, 16)] = zero16
                return 0

            lax.fori_loop(0, ACC_ROWS, init_row, 0)

            pltpu.sync_copy(
                cnt_hbm.at[pl.ds(pl.multiple_of(r * 16, 8), 16)], cvec)
            cn = jnp.max(cvec[...])
            nch = (cn + CEDGE - 1) // CEDGE

            def prep(c, dlc, tab=tab):
                sup = c // SUPC
                local = c - sup * SUPC

                @pl.when(local == 0)
                def _():
                    base = pl.multiple_of(r * CAP + sup * SUP, 8)
                    pltpu.sync_copy(srcl_hbm.at[pl.ds(base, SUP)], sidx_st)
                    pltpu.sync_copy(dll_hbm.at[pl.ds(base, SUP)], dlv_st)

                off = local * CEDGE
                for v16 in range(CEDGE // 16):
                    sl = pl.ds(v16 * 16, 16)
                    dlc[sl] = dlv_st[pl.ds(off + v16 * 16, 16)]
                return sidx_st.at[pl.ds(off, CEDGE)]

            def start_gather(c, gbuf, sem, dlc, tab=tab):
                idx = prep(c, dlc)
                return pltpu.async_copy(tab.at[idx], gbuf, sem)

            def compute(c, gbuf, dlc, t=t):
                def grp16(g, _2):
                    dls = dlc[pl.ds(g * 16, 16)]
                    dl_all = [jnp.max(jnp.where(iota == e16, dls, 0))
                              for e16 in range(16)]
                    for e16 in range(16):
                        dl = dl_all[e16]
                        e = g * 16 + e16
                        for j in range(D // 16):
                            sl = pl.ds(j * 16, 16)
                            v = gbuf[e, sl]
                            plsc.addupdate(asum.at[dl, sl], v)
                            plsc.addupdate(asq.at[dl, sl], v * v)
                            amn[dl, sl] = jnp.minimum(amn[dl, sl], v)
                            amx[dl, sl] = jnp.maximum(amx[dl, sl], v)
                        if t == 0:
                            plsc.addupdate(acnt.at[dl, pl.ds(0, 16)], ones)
                    return 0

                lax.fori_loop(0, CEDGE // 16, grp16, 0)

            @pl.when(nch > 0)
            def _():
                start_gather(0, gbufA, semA, dlcA)

            def pair_body(i, _):
                c0 = 2 * i
                c1 = c0 + 1
                pltpu.make_async_copy(tab.at[sidx_st.at[pl.ds(0, CEDGE)]],
                                      gbufA, semA).wait()

                @pl.when(c1 < nch)
                def _():
                    start_gather(c1, gbufB, semB, dlcB)

                compute(c0, gbufA, dlcA)

                @pl.when(c1 < nch)
                def _():
                    pltpu.make_async_copy(
                        tab.at[sidx_st.at[pl.ds(0, CEDGE)]],
                        gbufB, semB).wait()

                    @pl.when(c1 + 1 < nch)
                    def _():
                        start_gather(c1 + 1, gbufA, semA, dlcA)

                    compute(c1, gbufB, dlcB)
                return 0

            lax.fori_loop(0, (nch + 1) // 2, pair_body, 0)

            rows = pl.ds(pl.multiple_of(r * RN, 8), RN)
            pltpu.sync_copy(asum.at[pl.ds(0, RN)], s1.at[rows])
            pltpu.sync_copy(asq.at[pl.ds(0, RN)], s2.at[rows])
            pltpu.sync_copy(amn.at[pl.ds(0, RN)], mno.at[rows])
            pltpu.sync_copy(amx.at[pl.ds(0, RN)], mxo.at[rows])
            if t == 0:
                pltpu.sync_copy(acnt.at[pl.ds(0, RN)], cnto.at[rows])
            return 0

        lax.fori_loop(0, RPW, range_body, 0)


BN = 1000


def _pre_tc(h, wd, ws, bias):
    def body(h_ref, wd_ref, ws_ref, b_ref, a_ref, b0_ref, b1_ref):
        hb = h_ref[...]
        a_ref[...] = (
            jnp.dot(hb, wd_ref[...], preferred_element_type=jnp.float32)
            + b_ref[...]
        )
        bb = jnp.dot(hb, ws_ref[...], preferred_element_type=jnp.float32)
        b0_ref[...] = bb[:, :D]
        b1_ref[...] = bb[:, D:]

    return pl.pallas_call(
        body,
        grid=(N // BN,),
        in_specs=[
            pl.BlockSpec((BN, D), lambda i: (i, 0)),
            pl.BlockSpec((D, DF), lambda i: (0, 0)),
            pl.BlockSpec((D, DF), lambda i: (0, 0)),
            pl.BlockSpec((1, DF), lambda i: (0, 0)),
        ],
        out_specs=[
            pl.BlockSpec((BN, DF), lambda i: (i, 0)),
            pl.BlockSpec((BN, D), lambda i: (i, 0)),
            pl.BlockSpec((BN, D), lambda i: (i, 0)),
        ],
        out_shape=[
            jax.ShapeDtypeStruct((N, DF), jnp.float32),
            jax.ShapeDtypeStruct((N, D), jnp.float32),
            jax.ShapeDtypeStruct((N, D), jnp.float32),
        ],
        compiler_params=pltpu.CompilerParams(
            dimension_semantics=("arbitrary",)),
    )(h, wd, ws, bias)


def _post_tc(h, a_cat, aggs, cnt16, px, pr, pb, lw, lb):
    def body(h_ref, a_ref, s1a_ref, s2a_ref, mna_ref, mxa_ref,
             s1b_ref, s2b_ref, mnb_ref, mxb_ref, cnt_ref,
             px_ref, pr_ref, pb_ref, lw_ref, lb_ref,
             hp_ref, ps_ref, pq_ref):
        i = pl.program_id(0)
        cnt = cnt_ref[...][:, :1]
        cnt_c = jnp.maximum(cnt, 1.0)
        lg = jnp.log(cnt_c + 1.0)
        amp = lg / AVG_LOG
        att = AVG_LOG / lg
        has = cnt > 0.0
        hb = h_ref[...]
        tower_refs = ((s1a_ref, s2a_ref, mna_ref, mxa_ref),
                      (s1b_ref, s2b_ref, mnb_ref, mxb_ref))
        outs = []
        for t in range(2):
            fsl = pl.ds(t * D, D)
            osl = pl.ds(t * F_OUT, F_OUT)
            a = a_ref[:, fsl]
            s1_ref, s2_ref, mn_ref, mx_ref = tower_refs[t]
            ss1 = s1_ref[...]
            ss2 = s2_ref[...]
            ex1 = ss1 / cnt_c
            mean = cnt * a / cnt_c + ex1
            var = jnp.where(has, jnp.maximum(ss2 / cnt_c - ex1 * ex1, 0.0),
                            0.0)
            std = jnp.sqrt(var + 1e-5)
            mn_ = jnp.where(has, a + mn_ref[...], 0.0)
            mx_ = jnp.where(has, a + mx_ref[...], 0.0)
            agg = jnp.concatenate([mean, mn_, mx_, std], axis=-1)
            scaled = jnp.concatenate([agg, agg * amp, agg * att], axis=-1)
            o = (
                jnp.dot(hb, px_ref[:, osl],
                        preferred_element_type=jnp.float32)
                + jnp.dot(scaled, pr_ref[:, osl],
                          preferred_element_type=jnp.float32)
                + pb_ref[:, osl]
            )
            outs.append(o)
        oc = jnp.concatenate(outs, axis=-1)
        hp = jnp.dot(oc, lw_ref[...],
                     preferred_element_type=jnp.float32) + lb_ref[...]
        hp_ref[...] = hp

        @pl.when(i == 0)
        def _():
            ps_ref[...] = jnp.zeros_like(ps_ref)
            pq_ref[...] = jnp.zeros_like(pq_ref)

        ps_ref[...] += jnp.broadcast_to(
            jnp.sum(hp, axis=0, keepdims=True), ps_ref.shape)
        pq_ref[...] += jnp.broadcast_to(
            jnp.sum(hp * hp, axis=0, keepdims=True), pq_ref.shape)

    return pl.pallas_call(
        body,
        grid=(N // BN,),
        in_specs=[
            pl.BlockSpec((BN, D), lambda i: (i, 0)),
            pl.BlockSpec((BN, DF), lambda i: (i, 0)),
        ] + [pl.BlockSpec((BN, D), lambda i: (i, 0))] * 8 + [
            pl.BlockSpec((BN, 16), lambda i: (i, 0)),
            pl.BlockSpec((D, 2 * F_OUT), lambda i: (0, 0)),
            pl.BlockSpec((12 * D, 2 * F_OUT), lambda i: (0, 0)),
            pl.BlockSpec((1, 2 * F_OUT), lambda i: (0, 0)),
            pl.BlockSpec((D, D), lambda i: (0, 0)),
            pl.BlockSpec((1, D), lambda i: (0, 0)),
        ],
        out_specs=[
            pl.BlockSpec((BN, D), lambda i: (i, 0)),
            pl.BlockSpec((8, D), lambda i: (0, 0)),
            pl.BlockSpec((8, D), lambda i: (0, 0)),
        ],
        out_shape=[
            jax.ShapeDtypeStruct((N, D), jnp.float32),
            jax.ShapeDtypeStruct((8, D), jnp.float32),
            jax.ShapeDtypeStruct((8, D), jnp.float32),
        ],
        compiler_params=pltpu.CompilerParams(
            dimension_semantics=("arbitrary",)),
    )(h, a_cat, *aggs, cnt16, px, pr, pb, lw, lb)


def _norm_block(hp_ref, ps_ref, pq_ref, w_ref, b_ref, ms_ref):
    s = ps_ref[0:1, :]
    q = pq_ref[0:1, :]
    mean = s / N
    mm = mean * ms_ref[...]
    var = q / N - 2.0 * mm * mean + mm * mm
    hn = w_ref[...] * (hp_ref[...] - mm) / jnp.sqrt(var + 1e-5) + b_ref[...]
    return jnp.maximum(hn, 0.0)


def _norm_pre_tc(hp, ps, pq, gw, gb, gms, wd, ws, bias):
    def body(hp_ref, ps_ref, pq_ref, w_ref, b_ref, ms_ref,
             wd_ref, ws_ref, pb_ref, hn_ref, a_ref, b0_ref, b1_ref):
        h = _norm_block(hp_ref, ps_ref, pq_ref, w_ref, b_ref, ms_ref)
        hn_ref[...] = h
        a_ref[...] = (
            jnp.dot(h, wd_ref[...], preferred_element_type=jnp.float32)
            + pb_ref[...]
        )
        bb = jnp.dot(h, ws_ref[...], preferred_element_type=jnp.float32)
        b0_ref[...] = bb[:, :D]
        b1_ref[...] = bb[:, D:]

    return pl.pallas_call(
        body,
        grid=(N // BN,),
        in_specs=[
            pl.BlockSpec((BN, D), lambda i: (i, 0)),
            pl.BlockSpec((8, D), lambda i: (0, 0)),
            pl.BlockSpec((8, D), lambda i: (0, 0)),
            pl.BlockSpec((1, D), lambda i: (0, 0)),
            pl.BlockSpec((1, D), lambda i: (0, 0)),
            pl.BlockSpec((1, D), lambda i: (0, 0)),
            pl.BlockSpec((D, DF), lambda i: (0, 0)),
            pl.BlockSpec((D, DF), lambda i: (0, 0)),
            pl.BlockSpec((1, DF), lambda i: (0, 0)),
        ],
        out_specs=[
            pl.BlockSpec((BN, D), lambda i: (i, 0)),
            pl.BlockSpec((BN, DF), lambda i: (i, 0)),
            pl.BlockSpec((BN, D), lambda i: (i, 0)),
            pl.BlockSpec((BN, D), lambda i: (i, 0)),
        ],
        out_shape=[
            jax.ShapeDtypeStruct((N, D), jnp.float32),
            jax.ShapeDtypeStruct((N, DF), jnp.float32),
            jax.ShapeDtypeStruct((N, D), jnp.float32),
            jax.ShapeDtypeStruct((N, D), jnp.float32),
        ],
        compiler_params=pltpu.CompilerParams(
            dimension_semantics=("arbitrary",)),
    )(hp, ps, pq, gw, gb, gms, wd, ws, bias)


def _norm_fc_tc(hp, ps, pq, gw, gb, gms, w1, b1, w2, b2):
    def body(hp_ref, ps_ref, pq_ref, w_ref, b_ref, ms_ref,
             w1_ref, b1_ref, w2_ref, b2_ref, o_ref):
        h = _norm_block(hp_ref, ps_ref, pq_ref, w_ref, b_ref, ms_ref)
        h = jnp.maximum(
            jnp.dot(h, w1_ref[...], preferred_element_type=jnp.float32)
            + b1_ref[...], 0.0)
        o_ref[...] = (
            jnp.dot(h, w2_ref[...], preferred_element_type=jnp.float32)
            + b2_ref[...]
        )

    return pl.pallas_call(
        body,
        grid=(N // BN,),
        in_specs=[
            pl.BlockSpec((BN, D), lambda i: (i, 0)),
            pl.BlockSpec((8, D), lambda i: (0, 0)),
            pl.BlockSpec((8, D), lambda i: (0, 0)),
            pl.BlockSpec((1, D), lambda i: (0, 0)),
            pl.BlockSpec((1, D), lambda i: (0, 0)),
            pl.BlockSpec((1, D), lambda i: (0, 0)),
            pl.BlockSpec((D, D), lambda i: (0, 0)),
            pl.BlockSpec((1, D), lambda i: (0, 0)),
            pl.BlockSpec((D, D), lambda i: (0, 0)),
            pl.BlockSpec((1, D), lambda i: (0, 0)),
        ],
        out_specs=pl.BlockSpec((BN, D), lambda i: (i, 0)),
        out_shape=jax.ShapeDtypeStruct((N, D), jnp.float32),
        compiler_params=pltpu.CompilerParams(
            dimension_semantics=("arbitrary",)),
    )(hp, ps, pq, gw, gb, gms, w1, b1, w2, b2)


def kernel(x, edge_index, params):
    p = params
    src = edge_index[0].astype(jnp.int32)
    dst = edge_index[1].astype(jnp.int32)

    srcl, dll, cnts = _build_lists(dst, src)

    def pre_w(l):
        wd = jnp.concatenate(
            [p[f"pre_W_{l}_{t}"][:D] for t in range(2)], axis=1)
        ws = jnp.concatenate(
            [p[f"pre_W_{l}_{t}"][D:] for t in range(2)], axis=1)
        pb = jnp.concatenate(
            [p[f"pre_b_{l}_{t}"] for t in range(2)]).reshape(1, DF)
        return wd, ws, pb

    h = x
    out = None
    cnt16 = None
    for l in range(2):
        if l == 0:
            wd, ws, pb = pre_w(0)
            a_cat, b0, b1 = _pre_tc(h, wd, ws, pb)
        *aggs, cnt_new = _aggregate(b0, b1, srcl, dll, cnts)
        if cnt16 is None:
            cnt16 = cnt_new
        px = jnp.concatenate(
            [p[f"post_W_{l}_{t}"][:D] for t in range(2)], axis=1)
        pr = jnp.concatenate(
            [p[f"post_W_{l}_{t}"][D:] for t in range(2)], axis=1)
        pbo = jnp.concatenate(
            [p[f"post_b_{l}_{t}"] for t in range(2)]).reshape(1, 2 * F_OUT)
        hp, ps, pq = _post_tc(
            h, a_cat, aggs, cnt16, px, pr, pbo,
            p[f"lin_W_{l}"], p[f"lin_b_{l}"].reshape(1, D))
        gw = p[f"gn_w_{l}"].reshape(1, D)
        gb = p[f"gn_b_{l}"].reshape(1, D)
        gms = p[f"gn_ms_{l}"].reshape(1, D)
        if l == 0:
            wd, ws, pb = pre_w(1)
            h, a_cat, b0, b1 = _norm_pre_tc(hp, ps, pq, gw, gb, gms,
                                            wd, ws, pb)
        else:
            out = _norm_fc_tc(hp, ps, pq, gw, gb, gms,
                              p["fc1_W"], p["fc1_b"].reshape(1, D),
                              p["fc2_W"], p["fc2_b"].reshape(1, D))
    return out
